# baseline (speedup 1.0000x reference)
import functools

import jax
import jax.numpy as jnp
from jax import lax
from jax.experimental import pallas as pl
from jax.experimental.pallas import tpu as pltpu
from jax.experimental.pallas import tpu_sc as plsc


def _tc_body(kl_ref, ts_ref, idx_ref, rcp_ref, tso_ref):
    m = jnp.mean(kl_ref[...], axis=0)
    a = m[:, :64]
    b = m[:, 64:]
    a = a / jnp.sqrt(jnp.sum(a * a, axis=1, keepdims=True))
    b = b / jnp.sqrt(jnp.sum(b * b, axis=1, keepdims=True))
    scores = jax.lax.dot_general(a, b, (((1,), (1,)), ((), ())),
                                 precision=jax.lax.Precision.DEFAULT)
    node_max = jnp.max(scores, axis=1, keepdims=True)
    col = jax.lax.broadcasted_iota(jnp.int32, scores.shape, 1)
    row = jax.lax.broadcasted_iota(jnp.int32, scores.shape, 0)
    node_idx = jnp.min(jnp.where(scores == node_max, col, 1024),
                       axis=1, keepdims=True)
    p = jnp.where((col == node_idx) & (row > 0), 1.0, 0.0)
    ones = jnp.ones((1024, 1), dtype=jnp.float32)
    cnt = jax.lax.dot_general(p, ones, (((0,), (0,)), ((), ())),
                              precision=jax.lax.Precision.DEFAULT)
    ts_add = jax.lax.dot_general(p, ts_ref[:, :1], (((0,), (0,)), ((), ())),
                                 precision=jax.lax.Precision.DEFAULT)
    rid = jax.lax.broadcasted_iota(jnp.int32, (1024, 1), 0)
    idx_ref[...] = jnp.where(rid == 0, 1025, node_idx + 1)
    rcp_ref[...] = 1.0 / (1.0 + cnt)
    tso_ref[0:1, :] = ts_ref[0:1, 0:1]
    tso_ref[pl.ds(1, 1024), :] = ts_ref[:, 1:2] + ts_add


def _sc_body(hid_hbm, idx_hbm, rcp_hbm, out_hbm,
             idx_v, rcp_v, src0, src1, acc0, acc1):
    c = lax.axis_index("c")
    s = lax.axis_index("s")
    t = s * 2 + c

    @pl.when(t < 24)
    def _():
        c0 = t * 32
        pltpu.sync_copy(idx_hbm, idx_v)
        pltpu.sync_copy(rcp_hbm, rcp_v)
        pltpu.sync_copy(hid_hbm.at[:, pl.ds(c0, 16)], src0)
        pltpu.sync_copy(hid_hbm.at[:, pl.ds(c0 + 16, 16)], src1)
        pltpu.sync_copy(hid_hbm.at[pl.ds(0, 1), pl.ds(c0, 16)],
                        acc0.at[pl.ds(0, 1)])
        pltpu.sync_copy(hid_hbm.at[pl.ds(0, 1), pl.ds(c0 + 16, 16)],
                        acc1.at[pl.ds(0, 1)])
        pltpu.sync_copy(hid_hbm.at[:, pl.ds(768 + c0, 16)],
                        acc0.at[pl.ds(1, 1024)])
        pltpu.sync_copy(hid_hbm.at[:, pl.ds(768 + c0 + 16, 16)],
                        acc1.at[pl.ds(1, 1024)])

        def _accum(g, _):
            idx16 = idx_v[pl.ds(g * 16, 16)]
            for r in range(16):
                i = g * 16 + r
                d = idx16[r]
                acc0[d, :] = acc0[d, :] + src0[i, :]
                acc1[d, :] = acc1[d, :] + src1[i, :]
            return 0

        lax.fori_loop(0, 64, _accum, 0)

        def _scale(g, _):
            w16 = rcp_v[pl.ds(g * 16, 16)]
            for r in range(16):
                row = 1 + g * 16 + r
                w = jnp.full((16,), w16[r], dtype=jnp.float32)
                acc0[row, :] = acc0[row, :] * w
                acc1[row, :] = acc1[row, :] * w
            return 0

        lax.fori_loop(0, 64, _scale, 0)
        pltpu.sync_copy(acc0.at[pl.ds(0, 1025)], out_hbm.at[:, pl.ds(c0, 16)])
        pltpu.sync_copy(acc1.at[pl.ds(0, 1025)],
                        out_hbm.at[:, pl.ds(c0 + 16, 16)])


_sc_merge = functools.partial(
    pl.kernel,
    out_type=jax.ShapeDtypeStruct((1025, 768), jnp.float32),
    mesh=plsc.VectorSubcoreMesh(core_axis_name="c", subcore_axis_name="s"),
    compiler_params=pltpu.CompilerParams(use_tc_tiling_on_sc=False),
    scratch_types=[
        pltpu.VMEM((1024,), jnp.int32),
        pltpu.VMEM((1024,), jnp.float32),
        pltpu.VMEM((1024, 16), jnp.float32),
        pltpu.VMEM((1024, 16), jnp.float32),
        pltpu.VMEM((1026, 16), jnp.float32),
        pltpu.VMEM((1026, 16), jnp.float32),
    ],
)(_sc_body)


def kernel(hidden_states, attention_mask, self_attention_scores, key_layer,
           tome_size):
    del attention_mask, self_attention_scores
    kl = key_layer.reshape(12, 1024, 128)
    hid = hidden_states.reshape(1024, 1536)
    ts = tome_size.reshape(1024, 2)

    idx, rcp, ts_out = pl.pallas_call(
        _tc_body,
        out_shape=(
            jax.ShapeDtypeStruct((1024, 1), jnp.int32),
            jax.ShapeDtypeStruct((1024, 1), jnp.float32),
            jax.ShapeDtypeStruct((1025, 1), jnp.float32),
        ),
    )(kl, ts)

    out = _sc_merge(hid, idx.reshape(1024), rcp.reshape(1024))

    preserved = out[None]
    new_ts = ts_out[None]
    mask = jnp.zeros((1, 1, 1, 1025), dtype=hidden_states.dtype)
    return preserved, mask, new_ts


# speedup vs baseline: 1.3793x; 1.3793x over previous
import jax
import jax.numpy as jnp
from jax.experimental import pallas as pl


def _tome_body(kl_ref, hid_ref, ts_ref, out_ref, tso_ref):
    m = jnp.mean(kl_ref[...], axis=0)
    a = m[:, :64]
    b = m[:, 64:]
    a = a / jnp.sqrt(jnp.sum(a * a, axis=1, keepdims=True))
    b = b / jnp.sqrt(jnp.sum(b * b, axis=1, keepdims=True))
    scores = jax.lax.dot_general(a, b, (((1,), (1,)), ((), ())),
                                 precision=jax.lax.Precision.DEFAULT)
    node_max = jnp.max(scores, axis=1, keepdims=True)
    col = jax.lax.broadcasted_iota(jnp.int32, scores.shape, 1)
    row = jax.lax.broadcasted_iota(jnp.int32, scores.shape, 0)
    node_idx = jnp.min(jnp.where(scores == node_max, col, 1024),
                       axis=1, keepdims=True)
    p = jnp.where((col == node_idx) & (row > 0), 1.0, 0.0)
    he = hid_ref[:, :768]
    ho = hid_ref[:, 768:]
    add = jax.lax.dot_general(p, he, (((0,), (0,)), ((), ())),
                              precision=jax.lax.Precision.DEFAULT)
    ones = jnp.ones((1024, 1), dtype=jnp.float32)
    cnt = jax.lax.dot_general(p, ones, (((0,), (0,)), ((), ())),
                              precision=jax.lax.Precision.DEFAULT)
    ts_add = jax.lax.dot_general(p, ts_ref[:, :1], (((0,), (0,)), ((), ())),
                                 precision=jax.lax.Precision.DEFAULT)
    out_ref[...] = (ho + add) / (1.0 + cnt)
    tso_ref[...] = ts_ref[:, 1:2] + ts_add


def kernel(hidden_states, attention_mask, self_attention_scores, key_layer,
           tome_size):
    del attention_mask, self_attention_scores
    kl = key_layer.reshape(12, 1024, 128)
    hid = hidden_states.reshape(1024, 1536)
    ts = tome_size.reshape(1024, 2)

    out, ts_out = pl.pallas_call(
        _tome_body,
        out_shape=(
            jax.ShapeDtypeStruct((1024, 768), jnp.float32),
            jax.ShapeDtypeStruct((1024, 1), jnp.float32),
        ),
    )(kl, hid, ts)

    preserved = jnp.concatenate([hidden_states[:, :1, :], out[None]], axis=1)
    new_ts = jnp.concatenate([tome_size[:, :1, :], ts_out[None]], axis=1)
    mask = jnp.zeros((1, 1, 1, 1025), dtype=hidden_states.dtype)
    return preserved, mask, new_ts
